# baseline (speedup 1.0000x reference)
import functools

import jax
import jax.numpy as jnp
from jax import lax
from jax.experimental import pallas as pl
from jax.experimental.pallas import tpu as pltpu
from jax.experimental.pallas import tpu_sc as plsc

_TOP_K = 2
_AUX_W = 0.01

_NSC = 32
_HALF = 288
_CH = 72
_NW = 32
_R = 6


def _sc_pool_body(x_hbm, out_hbm, buf0, buf1, acc, sem0, sem1):
    dim = acc.shape[0]
    cid = lax.axis_index("c")
    sid = lax.axis_index("s")
    wid = sid * 2 + cid
    upt = (2 * _NSC) // _NW
    bufs = (buf0, buf1)
    sems = (sem0, sem1)
    nch = _HALF // _CH
    zero = jnp.zeros((16,), jnp.float32)
    for k in range(upt):
        u = wid * upt + k
        row = lax.rem(u, _NSC)
        half = u // _NSC
        base = half * _HALF
        handles = [None] * nch
        handles[0] = pltpu.async_copy(
            x_hbm.at[row, pl.ds(base, _CH)], bufs[0], sems[0])
        for d in range(dim // 16):
            acc[pl.ds(16 * d, 16)] = zero
        for ci in range(nch):
            if ci + 1 < nch:
                handles[ci + 1] = pltpu.async_copy(
                    x_hbm.at[row, pl.ds(base + (ci + 1) * _CH, _CH)],
                    bufs[(ci + 1) % 2], sems[(ci + 1) % 2])
            handles[ci].wait()
            cur = bufs[ci % 2]

            def body(i, carry, cur=cur):
                p = i * _R
                for g in range(0, dim // 16, 12):
                    vals = [[cur[p + r, pl.ds(16 * d, 16)]
                             for r in range(_R)]
                            for d in range(g, g + 12)]
                    for di, d in enumerate(range(g, g + 12)):
                        v = vals[di][0]
                        for r in range(1, _R):
                            v = v + vals[di][r]
                        plsc.addupdate(acc.at[pl.ds(16 * d, 16)], v)
                return carry

            lax.fori_loop(0, _CH // _R, body, 0)
        pltpu.sync_copy(acc, out_hbm.at[half, row])


def _tc_pool_body(x_ref, w_ref, out_ref, *, inv_hw):
    xb = x_ref[...]
    bb, hw, dim = xb.shape
    part = jnp.sum(xb.reshape(bb, hw // 8, 8, dim), axis=1)
    y = jax.lax.dot_general(
        part.reshape(bb * 8, dim), w_ref[...], (((1,), (1,)), ((), ())),
        preferred_element_type=jnp.float32, precision=jax.lax.Precision.HIGHEST)
    out_ref[...] = (jnp.sum(y.reshape(bb, 8, y.shape[-1]), axis=1)
                    * inv_hw)[None]


def _tail_body(part_ref, lb_ref, w_ref, aux_ref, idx_ref, score_ref, *,
               inv_hw):
    pooled_a = (part_ref[0] + part_ref[1]) * inv_hw
    logits_a = jax.lax.dot_general(
        pooled_a, w_ref[...], (((1,), (1,)), ((), ())),
        preferred_element_type=jnp.float32, precision=jax.lax.Precision.HIGHEST)
    logits = jnp.concatenate([logits_a, lb_ref[...]], axis=0)
    b, e = logits.shape
    m = jnp.max(logits, axis=-1, keepdims=True)
    ex = jnp.exp(logits - m)
    sm = ex / jnp.sum(ex, axis=-1, keepdims=True)

    col = jax.lax.broadcasted_iota(jnp.int32, (b, e), 1)
    s1 = jnp.max(sm, axis=-1, keepdims=True)
    idx1 = jnp.min(jnp.where(sm == s1, col, e), axis=-1, keepdims=True)
    masked = jnp.where(col == idx1, -jnp.inf, sm)
    s2 = jnp.max(masked, axis=-1, keepdims=True)
    idx2 = jnp.min(jnp.where(masked == s2, col, e), axis=-1, keepdims=True)

    onehot = ((col == idx1) | (col == idx2)).astype(jnp.float32)
    importance = jnp.mean(sm, axis=0)
    load = jnp.mean(onehot, axis=0) / _TOP_K
    aux_ref[...] = jnp.full(
        (1, 1), _AUX_W * float(e * e), jnp.float32) * jnp.mean(
            importance * load)

    k_col = jax.lax.broadcasted_iota(jnp.int32, (b, _TOP_K), 1)
    idx_ref[...] = jnp.where(k_col == 0, idx1, idx2).astype(jnp.int32)
    score_ref[...] = jnp.where(k_col == 0, s1, s2)


def kernel(x, W_gate):
    b, h, w, dim = x.shape
    e = W_gate.shape[0]
    hw = h * w
    x3 = x.reshape(b, hw, dim)
    inv_hw = 1.0 / hw

    sc_pool = functools.partial(
        pl.kernel,
        mesh=plsc.VectorSubcoreMesh(core_axis_name="c", subcore_axis_name="s"),
        out_type=jax.ShapeDtypeStruct((2, _NSC, dim), jnp.float32),
        scratch_types=[
            pltpu.VMEM((_CH, dim), jnp.float32),
            pltpu.VMEM((_CH, dim), jnp.float32),
            pltpu.VMEM((dim,), jnp.float32),
            pltpu.SemaphoreType.DMA,
            pltpu.SemaphoreType.DMA,
        ],
    )(_sc_pool_body)
    partials = sc_pool(x3)

    bb = 4
    logits_b = pl.pallas_call(
        functools.partial(_tc_pool_body, inv_hw=inv_hw),
        grid=((b - _NSC) // bb,),
        in_specs=[
            pl.BlockSpec((bb, hw, dim), lambda i: (i + _NSC // bb, 0, 0)),
            pl.BlockSpec((e, dim), lambda i: (0, 0)),
        ],
        out_specs=pl.BlockSpec((1, bb, e), lambda i: (i, 0, 0)),
        out_shape=jax.ShapeDtypeStruct(((b - _NSC) // bb, bb, e),
                                       jnp.float32),
        compiler_params=pltpu.CompilerParams(
            dimension_semantics=("arbitrary",)),
    )(x3, W_gate)

    aux, idx, scores = pl.pallas_call(
        functools.partial(_tail_body, inv_hw=inv_hw),
        out_shape=(
            jax.ShapeDtypeStruct((1, 1), jnp.float32),
            jax.ShapeDtypeStruct((b, _TOP_K), jnp.int32),
            jax.ShapeDtypeStruct((b, _TOP_K), jnp.float32),
        ),
    )(partials, logits_b.reshape(b - _NSC, e), W_gate)

    return aux.reshape(()), idx, scores


# speedup vs baseline: 1.2830x; 1.2830x over previous
import functools

import jax
import jax.numpy as jnp
from jax.experimental import pallas as pl
from jax.experimental.pallas import tpu as pltpu

_TOP_K = 2
_AUX_W = 0.01


def _fused_body(x_ref, w_ref, aux_ref, idx_ref, score_ref, logit_acc, *,
                inv_hw):
    i = pl.program_id(0)
    xb = x_ref[...]
    bb, hw, dim = xb.shape
    part = jnp.sum(xb.reshape(bb, hw // 8, 8, dim), axis=1)
    y = jax.lax.dot_general(
        part.reshape(bb * 8, dim), w_ref[...], (((1,), (1,)), ((), ())),
        preferred_element_type=jnp.float32,
        precision=jax.lax.Precision.HIGHEST)
    rows = jnp.sum(y.reshape(bb, 8, y.shape[-1]), axis=1) * inv_hw
    logit_acc[pl.ds(i * bb, bb), :] = rows

    @pl.when(i == pl.num_programs(0) - 1)
    def _finish():
        logits = logit_acc[...]
        b, e = logits.shape
        m = jnp.max(logits, axis=-1, keepdims=True)
        ex = jnp.exp(logits - m)
        sm = ex / jnp.sum(ex, axis=-1, keepdims=True)

        col = jax.lax.broadcasted_iota(jnp.int32, (b, e), 1)
        s1 = jnp.max(sm, axis=-1, keepdims=True)
        idx1 = jnp.min(jnp.where(sm == s1, col, e), axis=-1, keepdims=True)
        masked = jnp.where(col == idx1, -jnp.inf, sm)
        s2 = jnp.max(masked, axis=-1, keepdims=True)
        idx2 = jnp.min(jnp.where(masked == s2, col, e), axis=-1, keepdims=True)

        onehot = ((col == idx1) | (col == idx2)).astype(jnp.float32)
        importance = jnp.mean(sm, axis=0)
        load = jnp.mean(onehot, axis=0) / _TOP_K
        aux_ref[...] = jnp.full(
            (1, 1), _AUX_W * float(e * e), jnp.float32) * jnp.mean(
                importance * load)

        k_col = jax.lax.broadcasted_iota(jnp.int32, (b, _TOP_K), 1)
        idx_ref[...] = jnp.where(k_col == 0, idx1, idx2).astype(jnp.int32)
        score_ref[...] = jnp.where(k_col == 0, s1, s2)


def kernel(x, W_gate):
    b, h, w, dim = x.shape
    e = W_gate.shape[0]
    hw = h * w
    x3 = x.reshape(b, hw, dim)

    bb = 4
    grid = (b // bb,)

    aux, idx, scores = pl.pallas_call(
        functools.partial(_fused_body, inv_hw=1.0 / hw),
        grid=grid,
        in_specs=[
            pl.BlockSpec((bb, hw, dim), lambda i: (i, 0, 0)),
            pl.BlockSpec((e, dim), lambda i: (0, 0)),
        ],
        out_specs=(
            pl.BlockSpec((1, 1), lambda i: (0, 0)),
            pl.BlockSpec((b, _TOP_K), lambda i: (0, 0)),
            pl.BlockSpec((b, _TOP_K), lambda i: (0, 0)),
        ),
        out_shape=(
            jax.ShapeDtypeStruct((1, 1), jnp.float32),
            jax.ShapeDtypeStruct((b, _TOP_K), jnp.int32),
            jax.ShapeDtypeStruct((b, _TOP_K), jnp.float32),
        ),
        scratch_shapes=[pltpu.VMEM((b, e), jnp.float32)],
        compiler_params=pltpu.CompilerParams(
            dimension_semantics=("arbitrary",)),
    )(x3, W_gate)

    return aux.reshape(()), idx, scores
